# Initial kernel scaffold; baseline (speedup 1.0000x reference)
import jax
import jax.numpy as jnp
from jax.experimental import pallas as pl


def kernel(node_ids, neighbor_ids, reward, node_emd, bias_vector):
    raise NotImplementedError("write your pallas kernel here")


# speedup vs baseline: 1.1108x; 1.1108x over previous
import functools

import jax
import jax.numpy as jnp
from jax import lax
from jax.experimental import pallas as pl
from jax.experimental.pallas import tpu as pltpu
from jax.experimental.pallas import tpu_sc as plsc

LAMBDA_GEN = 1e-05
N_NODE = 100000
EMD_SIZE = 128
BATCH = 16384

_NC = 2
_NS = 16
_NW = _NC * _NS
_BPW = BATCH // _NW
_CHUNK = 128
_NCH = _BPW // _CHUNK


def _sc_gather_fn():
    mesh = plsc.VectorSubcoreMesh(core_axis_name="c", subcore_axis_name="s")

    @functools.partial(
        pl.kernel,
        out_type=[
            jax.ShapeDtypeStruct((BATCH, EMD_SIZE), jnp.float32),
            jax.ShapeDtypeStruct((BATCH, EMD_SIZE), jnp.float32),
            jax.ShapeDtypeStruct((BATCH,), jnp.float32),
        ],
        mesh=mesh,
        scratch_types=[
            pltpu.VMEM((_NCH, _CHUNK), jnp.int32),
            pltpu.VMEM((_NCH, _CHUNK), jnp.int32),
            pltpu.VMEM((_BPW, EMD_SIZE), jnp.float32),
            pltpu.VMEM((_BPW,), jnp.float32),
            pltpu.SemaphoreType.DMA,
        ],
    )
    def sc_gather(nids_hbm, vids_hbm, table_hbm, bias_hbm,
                  u_out, v_out, b_out,
                  nidx, vidx, rows, brows, sem):
        wid = lax.axis_index("s") * _NC + lax.axis_index("c")
        base = wid * _BPW
        pltpu.sync_copy(nids_hbm.at[pl.ds(wid * _NCH, _NCH)], nidx)
        pltpu.sync_copy(vids_hbm.at[pl.ds(wid * _NCH, _NCH)], vidx)
        cps = [
            pltpu.async_copy(table_hbm.at[nidx.at[j]],
                             rows.at[pl.ds(j * _CHUNK, _CHUNK)], sem)
            for j in range(_NCH)
        ]
        for c in cps:
            c.wait()
        pltpu.sync_copy(rows, u_out.at[pl.ds(base, _BPW)])
        cps = [
            pltpu.async_copy(table_hbm.at[vidx.at[j]],
                             rows.at[pl.ds(j * _CHUNK, _CHUNK)], sem)
            for j in range(_NCH)
        ]
        for c in cps:
            c.wait()
        pltpu.sync_copy(rows, v_out.at[pl.ds(base, _BPW)])
        cps = [
            pltpu.async_copy(bias_hbm.at[vidx.at[j]],
                             brows.at[pl.ds(j * _CHUNK, _CHUNK)], sem)
            for j in range(_NCH)
        ]
        for c in cps:
            c.wait()
        pltpu.sync_copy(brows, b_out.at[pl.ds(base, _BPW)])

    return sc_gather


_ROWS_PER_BLK = 2048
_GRID = BATCH // _ROWS_PER_BLK
_SUB = _ROWS_PER_BLK // 128


def _tc_loss_body(u_ref, v_ref, b_ref, r_ref, out_ref):
    i = pl.program_id(0)
    u = u_ref[...]
    v = v_ref[...]
    b = b_ref[...]
    dot = jnp.sum(u * v, axis=1).reshape(_SUB, 128)
    score = dot + b
    prob = jnp.clip(jax.nn.sigmoid(score), 1e-05, 1.0)
    data_term = jnp.sum(jnp.log(prob) * r_ref[...])
    l2 = 0.5 * (jnp.sum(u * u) + jnp.sum(v * v) + jnp.sum(b * b))
    part = -data_term / BATCH + LAMBDA_GEN * l2

    @pl.when(i == 0)
    def _():
        out_ref[0, 0] = 0.0

    out_ref[0, 0] += part


def _tc_loss(u, v, b2d, r2d):
    return pl.pallas_call(
        _tc_loss_body,
        grid=(_GRID,),
        in_specs=[
            pl.BlockSpec((_ROWS_PER_BLK, EMD_SIZE), lambda i: (i, 0)),
            pl.BlockSpec((_ROWS_PER_BLK, EMD_SIZE), lambda i: (i, 0)),
            pl.BlockSpec((_SUB, 128), lambda i: (i, 0)),
            pl.BlockSpec((_SUB, 128), lambda i: (i, 0)),
        ],
        out_specs=pl.BlockSpec(memory_space=pltpu.SMEM),
        out_shape=jax.ShapeDtypeStruct((1, 1), jnp.float32),
    )(u, v, b2d, r2d)


def kernel(node_ids, neighbor_ids, reward, node_emd, bias_vector):
    nids2d = node_ids.astype(jnp.int32).reshape(BATCH // _CHUNK, _CHUNK)
    vids2d = neighbor_ids.astype(jnp.int32).reshape(BATCH // _CHUNK, _CHUNK)
    u, v, bg = _sc_gather_fn()(nids2d, vids2d, node_emd, bias_vector)
    b2d = bg.reshape(BATCH // 128, 128)
    r2d = reward.reshape(BATCH // 128, 128)
    loss = _tc_loss(u, v, b2d, r2d)
    return loss[0, 0]


# speedup vs baseline: 1.2411x; 1.1173x over previous
import functools

import jax
import jax.numpy as jnp
from jax import lax
from jax.experimental import pallas as pl
from jax.experimental.pallas import tpu as pltpu
from jax.experimental.pallas import tpu_sc as plsc

LAMBDA_GEN = 1e-05
N_NODE = 100000
EMD_SIZE = 128
BATCH = 16384

_NC = 2
_NS = 16
_NW = _NC * _NS
_BPW = BATCH // _NW
_CHUNK = 128
_NCH = _BPW // _CHUNK
_L = 16
_NV = EMD_SIZE // _L


def _sc_fused_fn():
    mesh = plsc.VectorSubcoreMesh(core_axis_name="c", subcore_axis_name="s")

    @functools.partial(
        pl.kernel,
        out_type=[
            jax.ShapeDtypeStruct((BATCH,), jnp.float32),
            jax.ShapeDtypeStruct((_NW, _L), jnp.float32),
        ],
        mesh=mesh,
        compiler_params=pltpu.CompilerParams(needs_layout_passes=False),
        scratch_types=[
            pltpu.VMEM((_NCH, _CHUNK), jnp.int32),
            pltpu.VMEM((_NCH, _CHUNK), jnp.int32),
            pltpu.VMEM((2, _CHUNK, EMD_SIZE), jnp.float32),
            pltpu.VMEM((2, _CHUNK, EMD_SIZE), jnp.float32),
            pltpu.VMEM((_BPW,), jnp.float32),
            pltpu.VMEM((_BPW,), jnp.float32),
            pltpu.VMEM((_L,), jnp.float32),
            pltpu.SemaphoreType.DMA,
            pltpu.SemaphoreType.DMA,
        ],
    )
    def sc_fused(nids_hbm, vids_hbm, table_hbm, bias_hbm,
                 score_out, sq_out,
                 nidx, vidx, ubuf, vbuf, brows, score_buf, sqv, sem, bsem):
        wid = lax.axis_index("s") * _NC + lax.axis_index("c")
        base = wid * _BPW
        pltpu.sync_copy(nids_hbm.at[pl.ds(wid * _NCH, _NCH)], nidx)
        pltpu.sync_copy(vids_hbm.at[pl.ds(wid * _NCH, _NCH)], vidx)
        bias_cps = [
            pltpu.async_copy(bias_hbm.at[vidx.at[j]],
                             brows.at[pl.ds(j * _CHUNK, _CHUNK)], bsem)
            for j in range(_NCH)
        ]
        zeros = jnp.zeros((_L,), jnp.float32)
        for i in range(_BPW // _L):
            score_buf[pl.ds(i * _L, _L)] = zeros

        def fire(j):
            slot = j % 2
            return (
                pltpu.async_copy(table_hbm.at[nidx.at[j]], ubuf.at[slot], sem),
                pltpu.async_copy(table_hbm.at[vidx.at[j]], vbuf.at[slot], sem),
            )

        sq = jnp.zeros((_L,), jnp.float32)
        cps = fire(0)
        for j in range(_NCH):
            for c in cps:
                c.wait()
            if j + 1 < _NCH:
                cps = fire(j + 1)
            slot = j % 2
            u2d = ubuf.at[slot]
            v2d = vbuf.at[slot]
            sbase = j * _CHUNK

            def row_body(r, sq_acc):
                dot = None
                for k in range(_NV):
                    uk = u2d[r, pl.ds(k * _L, _L)]
                    vk = v2d[r, pl.ds(k * _L, _L)]
                    t = uk * vk
                    dot = t if dot is None else dot + t
                    sq_acc = sq_acc + uk * uk + vk * vk
                idxv = jnp.zeros((_L,), jnp.int32) + (sbase + r)
                plsc.addupdate_scatter(score_buf, [idxv], dot)
                return sq_acc

            sq = lax.fori_loop(0, _CHUNK, row_body, sq, unroll=2)

        for c in bias_cps:
            c.wait()
        for i in range(_BPW // _L):
            bv = brows[pl.ds(i * _L, _L)]
            score_buf[pl.ds(i * _L, _L)] += bv
            sq = sq + bv * bv

        sqv[...] = sq
        pltpu.sync_copy(score_buf, score_out.at[pl.ds(base, _BPW)])
        pltpu.sync_copy(sqv, sq_out.at[wid])

    return sc_fused


def _tc_loss_body(s_ref, r_ref, sq_ref, out_ref):
    score = s_ref[...]
    prob = jnp.clip(jax.nn.sigmoid(score), 1e-05, 1.0)
    data_term = jnp.sum(jnp.log(prob) * r_ref[...])
    l2 = 0.5 * jnp.sum(sq_ref[...])
    out_ref[0, 0] = -data_term / BATCH + LAMBDA_GEN * l2


def _tc_loss(s2d, r2d, sq):
    return pl.pallas_call(
        _tc_loss_body,
        out_specs=pl.BlockSpec(memory_space=pltpu.SMEM),
        out_shape=jax.ShapeDtypeStruct((1, 1), jnp.float32),
    )(s2d, r2d, sq)


def kernel(node_ids, neighbor_ids, reward, node_emd, bias_vector):
    nids2d = node_ids.astype(jnp.int32).reshape(BATCH // _CHUNK, _CHUNK)
    vids2d = neighbor_ids.astype(jnp.int32).reshape(BATCH // _CHUNK, _CHUNK)
    score, sq = _sc_fused_fn()(nids2d, vids2d, node_emd, bias_vector)
    s2d = score.reshape(BATCH // 128, 128)
    r2d = reward.reshape(BATCH // 128, 128)
    loss = _tc_loss(s2d, r2d, sq)
    return loss[0, 0]
